# baseline (speedup 1.0000x reference)
import functools

import jax
import jax.numpy as jnp
from jax import lax
from jax.experimental import pallas as pl
from jax.experimental.pallas import tpu as pltpu
from jax.experimental.pallas import tpu_sc as plsc

_NUM_FIXED = 998976
_NUM_TRAINABLE = 1024
_WV_DIM = 32

_NUM_WORKERS = 32
_HIST = 200
_ROWS_PER_SLOT = 2
_LANES = 16
_NBUF = 4
_GROUPS_PER_ROW = 13
_PIECES = ((0, 128), (128, 72))


def _emb_body(idx_hbm, fixed_hbm, train_hbm, out_hbm,
              idx_raw, idx_dma, rows, train_v, acc_v, *sems):
    sem_i = sems[0:_NBUF]
    sem_g = sems[_NBUF:2 * _NBUF]
    sem_o = sems[2 * _NBUF:3 * _NBUF]

    num_cores = 2
    wid = lax.axis_index("s") * num_cores + lax.axis_index("c")
    rows_per_worker = idx_hbm.shape[0] // _NUM_WORKERS
    num_steps = rows_per_worker // _ROWS_PER_SLOT
    worker_row0 = wid * rows_per_worker

    ones = jnp.full((_LANES,), 1, jnp.int32)
    zeros = jnp.full((_LANES,), 0, jnp.int32)

    pltpu.sync_copy(train_hbm, train_v)

    def fire_idx(g, b):
        r0 = worker_row0 + g * _ROWS_PER_SLOT
        pltpu.async_copy(idx_hbm.at[pl.ds(r0, _ROWS_PER_SLOT)],
                         idx_raw.at[b], sem_i[b])

    def wait_idx(b):
        pltpu.make_async_copy(idx_hbm.at[pl.ds(0, _ROWS_PER_SLOT)],
                              idx_raw.at[b], sem_i[b]).wait()

    def clamp(b):
        def clamp_body(j, c0):
            off = jnp.minimum(j * _LANES, _HIST - _LANES)
            for i in range(_ROWS_PER_SLOT):
                v = idx_raw[b, i, pl.ds(off, _LANES)]
                m = v >= _NUM_FIXED
                idx_dma[b, i, pl.ds(off, _LANES)] = (
                    jnp.where(m, _NUM_FIXED - 1, v))
                acc_v[b, :] = acc_v[b, :] + jnp.where(m, ones, zeros)
            return c0

        acc_v[b, :] = zeros
        lax.fori_loop(0, _GROUPS_PER_ROW, clamp_body, 0)

    def fire_gather(b):
        for i in range(_ROWS_PER_SLOT):
            for off, ln in _PIECES:
                pltpu.async_copy(
                    fixed_hbm.at[idx_dma.at[b, i, pl.ds(off, ln)]],
                    rows.at[b, i, pl.ds(off, ln)],
                    sem_g[b])

    def wait_gather(b):
        pltpu.make_async_copy(rows.at[b],
                              out_hbm.at[pl.ds(0, _ROWS_PER_SLOT)],
                              sem_g[b]).wait()

    def fire_out(g, b):
        r0 = worker_row0 + g * _ROWS_PER_SLOT
        pltpu.async_copy(rows.at[b], out_hbm.at[pl.ds(r0, _ROWS_PER_SLOT)],
                         sem_o[b])

    def wait_out(b):
        pltpu.make_async_copy(rows.at[b],
                              out_hbm.at[pl.ds(0, _ROWS_PER_SLOT)],
                              sem_o[b]).wait()

    def patch(b):
        step_hits = jnp.sum(acc_v[b, :])

        @pl.when(step_hits > 0)
        def _patch():
            def patch_body(j, c1):
                off = jnp.minimum(j * _LANES, _HIST - _LANES)
                for i in range(_ROWS_PER_SLOT):
                    v = idx_raw[b, i, pl.ds(off, _LANES)]
                    m = v >= _NUM_FIXED
                    cnt = jnp.sum(jnp.where(m, ones, zeros))

                    @pl.when(cnt > 0)
                    def _():
                        t = jnp.maximum(v - _NUM_FIXED, 0)
                        h = off + lax.iota(jnp.int32, _LANES)
                        ivec = jnp.full((_LANES,), i, jnp.int32)
                        for c in range(_WV_DIM):
                            cvec = jnp.full((_LANES,), c, jnp.int32)
                            vals = plsc.load_gather(train_v, [t, cvec],
                                                    mask=m)
                            plsc.store_scatter(rows.at[b], [ivec, h, cvec],
                                               vals, mask=m)

                return c1

            lax.fori_loop(0, _GROUPS_PER_ROW, patch_body, 0)

    fire_idx(0, 0)
    fire_idx(1, 1)
    wait_idx(0)
    clamp(0)
    fire_gather(0)

    def group_body(gg, carry):
        for b0 in range(_NBUF):
            g = gg * _NBUF + b0
            b1 = (b0 + 1) % _NBUF
            b2 = (b0 + 2) % _NBUF

            @pl.when(g + 2 < num_steps)
            def _():
                fire_idx(g + 2, b2)

            @pl.when(g + 1 < num_steps)
            def _():
                wait_idx(b1)
                clamp(b1)

                @pl.when(g + 1 >= _NBUF)
                def _():
                    wait_out(b1)

                fire_gather(b1)

            wait_gather(b0)
            patch(b0)
            fire_out(g, b0)
        return carry

    lax.fori_loop(0, num_steps // _NBUF, group_body, 0)

    for b in range(_NBUF):
        wait_out(b)


def kernel(inputs, fixed_wv_weight, trainable_wv_weight):
    batch, hist = inputs.shape

    mesh = plsc.VectorSubcoreMesh(core_axis_name="c", subcore_axis_name="s")
    run = functools.partial(
        pl.kernel,
        out_type=jax.ShapeDtypeStruct((batch, hist, _WV_DIM), jnp.float32),
        mesh=mesh,
        compiler_params=pltpu.CompilerParams(use_tc_tiling_on_sc=False,
                                             needs_layout_passes=False),
        scratch_types=[
            pltpu.VMEM((_NBUF, _ROWS_PER_SLOT, _HIST), jnp.int32),
            pltpu.VMEM((_NBUF, _ROWS_PER_SLOT, _HIST), jnp.int32),
            pltpu.VMEM((_NBUF, _ROWS_PER_SLOT, _HIST, _WV_DIM),
                       jnp.float32),
            pltpu.VMEM((_NUM_TRAINABLE, _WV_DIM), jnp.float32),
            pltpu.VMEM((_NBUF, _LANES), jnp.int32),
        ] + [pltpu.SemaphoreType.DMA] * (3 * _NBUF),
    )(_emb_body)
    return run(inputs, fixed_wv_weight, trainable_wv_weight)


# speedup vs baseline: 1.9839x; 1.9839x over previous
import functools

import jax
import jax.numpy as jnp
from jax import lax
from jax.experimental import pallas as pl
from jax.experimental.pallas import tpu as pltpu
from jax.experimental.pallas import tpu_sc as plsc

_NUM_FIXED = 998976
_NUM_TRAINABLE = 1024
_WV_DIM = 32

_NUM_WORKERS = 32
_LANES = 16
_NBUF = 2
_TH = 25
_TB = 128
_SUB = 4
_BL = 128
_CT = _WV_DIM // 8
_TPAD = _BL + 1


def _emb_body(idx_hbm, fixed_hbm, train_hbm, out_hbm,
              idx_raw, idx_dma, stage, tstage, train_v, acc_v, *sems):
    sem_i = sems[0:_NBUF]
    sem_g = sems[_NBUF:2 * _NBUF]
    sem_o = sems[2 * _NBUF:3 * _NBUF]

    num_cores = 2
    wid = lax.axis_index("s") * num_cores + lax.axis_index("c")
    num_units = (_TH * _TB * 2) // _NUM_WORKERS
    u0 = wid * num_units

    ones = jnp.full((_LANES,), 1, jnp.int32)
    zeros = jnp.full((_LANES,), 0, jnp.int32)

    pltpu.sync_copy(train_hbm, train_v)

    def unit_coords(u):
        t_h = u // (_TB * 2)
        rem = u % (_TB * 2)
        t_b = rem // 2
        sh = rem % 2
        return t_h, t_b, sh

    def fire_idx(u, bb):
        t_h, t_b, sh = unit_coords(u)
        pltpu.async_copy(idx_hbm.at[t_h, t_b, pl.ds(sh * _SUB, _SUB)],
                         idx_raw.at[bb], sem_i[bb])

    def wait_idx(bb):
        pltpu.make_async_copy(idx_hbm.at[0, 0, pl.ds(0, _SUB)],
                              idx_raw.at[bb], sem_i[bb]).wait()

    def clamp(bb):
        acc_v[bb, :] = zeros
        for s in range(_SUB):
            def clamp_body(j, c0, s=s):
                v = idx_raw[bb, s, pl.ds(j * _LANES, _LANES)]
                m = v >= _NUM_FIXED
                idx_dma[bb, s, pl.ds(j * _LANES, _LANES)] = (
                    jnp.where(m, _NUM_FIXED - 1, v))
                acc_v[bb, :] = acc_v[bb, :] + jnp.where(m, ones, zeros)
                return c0

            lax.fori_loop(0, _BL // _LANES, clamp_body, 0)

    def fire_gather(bb):
        for s in range(_SUB):
            pltpu.async_copy(fixed_hbm.at[idx_dma.at[bb, s]],
                             stage.at[bb, s], sem_g[bb])

    def wait_gather(bb):
        for s in range(_SUB):
            pltpu.make_async_copy(fixed_hbm.at[pl.ds(0, _BL)],
                                  stage.at[bb, s], sem_g[bb]).wait()

    def transpose_slab(bb, s):
        c_lo = lax.iota(jnp.int32, _LANES)
        c_hi = c_lo + _LANES

        def tr_body(j, c0):
            r0 = j * 4
            for dr in range(4):
                blvec = jnp.full((_LANES,), dr, jnp.int32) + r0
                lo = stage[bb, s, r0 + dr, pl.ds(0, _LANES)]
                hi = stage[bb, s, r0 + dr, pl.ds(_LANES, _LANES)]
                plsc.store_scatter(tstage.at[bb, s], [c_lo, blvec], lo)
                plsc.store_scatter(tstage.at[bb, s], [c_hi, blvec], hi)
            return c0

        lax.fori_loop(0, _BL // 4, tr_body, 0)

    def fire_out_slab(u, bb, s):
        t_h, t_b, sh = unit_coords(u)
        h0 = t_h * 8 + sh * _SUB
        for tc in range(_CT):
            pltpu.async_copy(
                tstage.at[bb, s, pl.ds(tc * 8, 8), pl.ds(0, _BL)],
                out_hbm.at[h0 + s, tc, t_b],
                sem_o[bb])

    def wait_out(bb):
        for s in range(_SUB):
            for tc in range(_CT):
                pltpu.make_async_copy(
                    tstage.at[bb, s, pl.ds(tc * 8, 8), pl.ds(0, _BL)],
                    out_hbm.at[0, 0, 0],
                    sem_o[bb]).wait()

    def patch_slab(bb, s, unit_hits):
        @pl.when(unit_hits > 0)
        def _patch():
            def patch_body(j, c1):
                p = j * _LANES
                v = idx_raw[bb, s, pl.ds(p, _LANES)]
                m = v >= _NUM_FIXED
                cnt = jnp.sum(jnp.where(m, ones, zeros))

                @pl.when(cnt > 0)
                def _():
                    t = jnp.maximum(v - _NUM_FIXED, 0)
                    r = p + lax.iota(jnp.int32, _LANES)
                    for c in range(_WV_DIM):
                        cvec = jnp.full((_LANES,), c, jnp.int32)
                        vals = plsc.load_gather(train_v, [t, cvec], mask=m)
                        plsc.store_scatter(stage.at[bb, s],
                                           [r, cvec], vals, mask=m)

                return c1

            lax.fori_loop(0, _BL // _LANES, patch_body, 0)

    fire_idx(u0, 0)
    fire_idx(u0 + 1, 1)
    wait_idx(0)
    clamp(0)
    fire_gather(0)

    def group_body(gg, carry):
        for b0 in range(_NBUF):
            g = gg * _NBUF + b0
            u = u0 + g
            b1 = (b0 + 1) % _NBUF

            @pl.when(g + 1 < num_units)
            def _():
                wait_idx(b1)
                clamp(b1)
                fire_gather(b1)

            wait_gather(b0)
            unit_hits = jnp.sum(acc_v[b0, :])
            for s in range(_SUB):
                patch_slab(b0, s, unit_hits)

            @pl.when(g + 2 < num_units)
            def _():
                fire_idx(u + 2, b0)

            @pl.when(g >= 2)
            def _():
                wait_out(b0)

            for s in range(_SUB):
                transpose_slab(b0, s)
            for s in range(_SUB):
                fire_out_slab(u, b0, s)
        return carry

    lax.fori_loop(0, num_units // _NBUF, group_body, 0)

    for b in range(_NBUF):
        wait_out(b)


def kernel(inputs, fixed_wv_weight, trainable_wv_weight):
    batch, hist = inputs.shape

    idx_view = (inputs.T.reshape(_TH, 8, _TB, _BL)
                .transpose(0, 2, 1, 3))

    mesh = plsc.VectorSubcoreMesh(core_axis_name="c", subcore_axis_name="s")
    run = functools.partial(
        pl.kernel,
        out_type=jax.ShapeDtypeStruct((hist, _CT, _TB, 8, _BL), jnp.float32),
        mesh=mesh,
        compiler_params=pltpu.CompilerParams(use_tc_tiling_on_sc=False,
                                             needs_layout_passes=False),
        scratch_types=[
            pltpu.VMEM((_NBUF, _SUB, _BL), jnp.int32),
            pltpu.VMEM((_NBUF, _SUB, _BL), jnp.int32),
            pltpu.VMEM((_NBUF, _SUB, _BL, _WV_DIM), jnp.float32),
            pltpu.VMEM((_NBUF, _SUB, _WV_DIM, _TPAD), jnp.float32),
            pltpu.VMEM((_NUM_TRAINABLE, _WV_DIM), jnp.float32),
            pltpu.VMEM((_NBUF, _LANES), jnp.int32),
        ] + [pltpu.SemaphoreType.DMA] * (3 * _NBUF),
    )(_emb_body)
    out_view = run(idx_view, fixed_wv_weight, trainable_wv_weight)
    return (out_view.transpose(2, 4, 0, 1, 3)
            .reshape(batch, hist, _WV_DIM))
